# baseline (speedup 1.0000x reference)
import jax
import jax.numpy as jnp
from jax.experimental import pallas as pl
from jax.experimental.pallas import tpu as pltpu

N = 2048
H = 6
BR = 1024
NR = N // BR
MASK_OFF = -1080.0


def _mask_prep(a_int, a_nh):
    def body(ai_ref, an_ref, m_ref):
        b = pl.program_id(0)
        i = pl.program_id(1)
        rows = jax.lax.broadcasted_iota(jnp.int32, (BR, N), 0) + i * BR
        cols = jax.lax.broadcasted_iota(jnp.int32, (BR, N), 1)
        sel = jnp.where(b == 0, ai_ref[...], an_ref[...])
        m = (sel > 0.99) | (rows == cols)
        m_ref[0] = jnp.where(m, 0.0, MASK_OFF).astype(jnp.bfloat16)

    return pl.pallas_call(
        body,
        grid=(2, NR),
        in_specs=[
            pl.BlockSpec((BR, N), lambda b, i: (i * (1 - b), 0)),
            pl.BlockSpec((BR, N), lambda b, i: (i * b, 0)),
        ],
        out_specs=pl.BlockSpec((1, BR, N), lambda b, i: (b, i, 0)),
        out_shape=jax.ShapeDtypeStruct((2, N, N), jnp.bfloat16),
    )(a_int, a_nh)


def _gat(x, masks, W, ALR, AR):
    Bx, _, Fin = x.shape
    Fo = W.shape[-1]

    def body(x_ref, w_ref, alr_ref, ar_ref, m_ref, o_ref,
             h_scr, b1_scr, b2_scr, fr_scr, fr2_scr):
        bb = pl.program_id(0)
        i = pl.program_id(1)

        @pl.when((bb == 0) & (i == 0))
        def _ones():
            for g in range(H):
                h_scr[g, :, Fo:] = jnp.ones((N, 1), jnp.float32)

        @pl.when(i == 0)
        def _project():
            for g in range(H):
                acc = jnp.dot(x_ref[0], w_ref[0, g],
                              preferred_element_type=jnp.float32)
                h_scr[g, :, :Fo] = acc
                fld = jnp.dot(acc, alr_ref[0, g],
                              preferred_element_type=jnp.float32)
                frr = jax.lax.dot_general(
                    ar_ref[0, g], acc, (((1,), (1,)), ((), ())),
                    preferred_element_type=jnp.float32)
                fl = fld[:, 0:1]
                fd = fl + fld[:, 1:2]
                d = jnp.maximum(fd, 0.2 * fd)
                b1_scr[g] = fl - d
                b2_scr[g] = 0.2 * fl - d
                fr_scr[g] = frr
                fr2_scr[g] = 0.2 * frr

        madd = m_ref[0].astype(jnp.float32)
        for g in range(H):
            b1 = b1_scr[g, pl.ds(i * BR, BR), :]
            b2 = b2_scr[g, pl.ds(i * BR, BR), :]
            t = jnp.maximum(fr_scr[g] + b1, fr2_scr[g] + b2)
            p = jnp.exp2(t + madd)
            oext = jnp.dot(p, h_scr[g], preferred_element_type=jnp.float32)
            o = oext[:, :Fo] / oext[:, Fo:]
            o_ref[0, :, g * Fo:(g + 1) * Fo] = jnp.maximum(o, 0.0)

    xmap = (lambda b, i: (b, 0, 0)) if Bx == 2 else (lambda b, i: (0, 0, 0))
    return pl.pallas_call(
        body,
        grid=(2, NR),
        in_specs=[
            pl.BlockSpec((1, N, Fin), xmap),
            pl.BlockSpec((1, H, Fin, Fo), lambda b, i: (b, 0, 0, 0)),
            pl.BlockSpec((1, H, Fo, 2), lambda b, i: (b, 0, 0, 0)),
            pl.BlockSpec((1, H, 1, Fo), lambda b, i: (b, 0, 0, 0)),
            pl.BlockSpec((1, BR, N), lambda b, i: (b, i, 0)),
        ],
        out_specs=pl.BlockSpec((1, BR, H * Fo), lambda b, i: (b, i, 0)),
        out_shape=jax.ShapeDtypeStruct((2, N, H * Fo), jnp.float32),
        scratch_shapes=[
            pltpu.VMEM((H, N, Fo + 1), jnp.float32),
            pltpu.VMEM((H, N, 1), jnp.float32),
            pltpu.VMEM((H, N, 1), jnp.float32),
            pltpu.VMEM((H, 1, N), jnp.float32),
            pltpu.VMEM((H, 1, N), jnp.float32),
        ],
    )(x, W, ALR, AR, masks)


def _gat_final(x, masks, W, ALR, AR, wd, bd):
    _, _, Fin = x.shape
    Fo = W.shape[-1]

    def body(x_ref, w_ref, alr_ref, ar_ref, m_ref, wd_ref, wd2_ref, bd_ref,
             o_ref, h_scr, b1_scr, b2_scr, fr_scr, fr2_scr, s_scr):
        bb = pl.program_id(0)
        i = pl.program_id(1)

        @pl.when((bb == 0) & (i == 0))
        def _ones():
            for g in range(H):
                h_scr[g, :, Fo:] = jnp.ones((N, 1), jnp.float32)
            s_scr[...] = jnp.zeros((2, 1, H * Fo), jnp.float32)

        @pl.when(i == 0)
        def _project():
            for g in range(H):
                acc = jnp.dot(x_ref[0], w_ref[0, g],
                              preferred_element_type=jnp.float32)
                h_scr[g, :, :Fo] = acc
                fld = jnp.dot(acc, alr_ref[0, g],
                              preferred_element_type=jnp.float32)
                frr = jax.lax.dot_general(
                    ar_ref[0, g], acc, (((1,), (1,)), ((), ())),
                    preferred_element_type=jnp.float32)
                fl = fld[:, 0:1]
                fd = fl + fld[:, 1:2]
                d = jnp.maximum(fd, 0.2 * fd)
                b1_scr[g] = fl - d
                b2_scr[g] = 0.2 * fl - d
                fr_scr[g] = frr
                fr2_scr[g] = 0.2 * frr

        madd = m_ref[0].astype(jnp.float32)
        cols = []
        for g in range(H):
            b1 = b1_scr[g, pl.ds(i * BR, BR), :]
            b2 = b2_scr[g, pl.ds(i * BR, BR), :]
            t = jnp.maximum(fr_scr[g] + b1, fr2_scr[g] + b2)
            p = jnp.exp2(t + madd)
            oext = jnp.dot(p, h_scr[g], preferred_element_type=jnp.float32)
            o = jnp.maximum(oext[:, :Fo] / oext[:, Fo:], 0.0)
            cols.append(jnp.sum(o, axis=0, keepdims=True))
        part = jnp.concatenate(cols, axis=1)

        @pl.when(bb == 0)
        def _acc0():
            s_scr[0] = s_scr[0] + part

        @pl.when(bb == 1)
        def _acc1():
            s_scr[1] = s_scr[1] + part

        @pl.when((bb == 1) & (i == NR - 1))
        def _flush():
            def tot(z):
                return jnp.sum(z, axis=(0, 1), keepdims=True)
            s0 = s_scr[0]
            s1 = s_scr[1]
            n2 = tot(s0 * s0) + tot(s1 * s1)
            num = tot(s0 * wd_ref[0]) + tot(s1 * wd2_ref[0])
            nrm = jnp.maximum(jnp.sqrt(n2), jnp.float32(1e-12))
            o_ref[...] = num / nrm + bd_ref[...]

    return pl.pallas_call(
        body,
        grid=(2, NR),
        in_specs=[
            pl.BlockSpec((1, N, Fin), lambda b, i: (b, 0, 0)),
            pl.BlockSpec((1, H, Fin, Fo), lambda b, i: (b, 0, 0, 0)),
            pl.BlockSpec((1, H, Fo, 2), lambda b, i: (b, 0, 0, 0)),
            pl.BlockSpec((1, H, 1, Fo), lambda b, i: (b, 0, 0, 0)),
            pl.BlockSpec((1, BR, N), lambda b, i: (b, i, 0)),
            pl.BlockSpec((1, 1, H * Fo), lambda b, i: (0, 0, 0)),
            pl.BlockSpec((1, 1, H * Fo), lambda b, i: (1, 0, 0)),
            pl.BlockSpec((1, 1), lambda b, i: (0, 0)),
        ],
        out_specs=pl.BlockSpec((1, 1), lambda b, i: (0, 0)),
        out_shape=jax.ShapeDtypeStruct((1, 1), jnp.float32),
        scratch_shapes=[
            pltpu.VMEM((H, N, Fo + 1), jnp.float32),
            pltpu.VMEM((H, N, 1), jnp.float32),
            pltpu.VMEM((H, N, 1), jnp.float32),
            pltpu.VMEM((H, 1, N), jnp.float32),
            pltpu.VMEM((H, 1, N), jnp.float32),
            pltpu.VMEM((2, 1, H * Fo), jnp.float32),
        ],
    )(x, W, ALR, AR, masks, wd, wd, bd)


def _att_prep(Ai, An, Fo):
    log2e = jnp.float32(1.4426950408889634)
    A2 = jnp.stack([Ai, An]) * log2e
    alr = jnp.stack([A2[:, :, :Fo], A2[:, :, Fo:]], axis=-1)
    ar = A2[:, :, None, Fo:]
    return alr, ar


def kernel(v, a_int, a_nh, W1i, A1i, W1n, A1n, W2i, A2i, W2n, A2n,
           W3i, A3i, W3n, A3n, Wd, bd):
    masks = _mask_prep(a_int, a_nh)

    alr1, ar1 = _att_prep(A1i, A1n, 16)
    h = _gat(v[None], masks, jnp.stack([W1i, W1n]), alr1, ar1)
    alr2, ar2 = _att_prep(A2i, A2n, 32)
    h = _gat(h, masks, jnp.stack([W2i, W2n]), alr2, ar2)
    alr3, ar3 = _att_prep(A3i, A3n, 64)
    out = _gat_final(h, masks, jnp.stack([W3i, W3n]), alr3, ar3,
                     Wd[:, 0].reshape(2, 1, H * 64), bd.reshape(1, 1))
    return out.reshape(1)


# speedup vs baseline: 1.0139x; 1.0139x over previous
import jax
import jax.numpy as jnp
from jax.experimental import pallas as pl
from jax.experimental.pallas import tpu as pltpu

N = 2048
H = 6
BR = 512
NR = N // BR
MASK_OFF = -1080.0


def _mask_prep(a_int, a_nh):
    def body(ai_ref, an_ref, m_ref):
        b = pl.program_id(0)
        i = pl.program_id(1)
        rows = jax.lax.broadcasted_iota(jnp.int32, (BR, N), 0) + i * BR
        cols = jax.lax.broadcasted_iota(jnp.int32, (BR, N), 1)
        sel = jnp.where(b == 0, ai_ref[...], an_ref[...])
        m = (sel > 0.99) | (rows == cols)
        m_ref[0] = jnp.where(m, 0.0, MASK_OFF).astype(jnp.bfloat16)

    return pl.pallas_call(
        body,
        grid=(2, NR),
        in_specs=[
            pl.BlockSpec((BR, N), lambda b, i: (i * (1 - b), 0)),
            pl.BlockSpec((BR, N), lambda b, i: (i * b, 0)),
        ],
        out_specs=pl.BlockSpec((1, BR, N), lambda b, i: (b, i, 0)),
        out_shape=jax.ShapeDtypeStruct((2, N, N), jnp.bfloat16),
    )(a_int, a_nh)


def _gat(x, masks, W, ALR, AR):
    Bx, _, Fin = x.shape
    Fo = W.shape[-1]

    def body(x_ref, w_ref, alr_ref, ar_ref, m_ref, o_ref,
             h_scr, b1_scr, b2_scr, fr_scr, fr2_scr):
        bb = pl.program_id(0)
        i = pl.program_id(1)

        @pl.when((bb == 0) & (i == 0))
        def _ones():
            for g in range(H):
                h_scr[g, :, Fo:] = jnp.ones((N, 1), jnp.float32)

        @pl.when(i == 0)
        def _project():
            for g in range(H):
                acc = jnp.dot(x_ref[0], w_ref[0, g],
                              preferred_element_type=jnp.float32)
                h_scr[g, :, :Fo] = acc
                fld = jnp.dot(acc, alr_ref[0, g],
                              preferred_element_type=jnp.float32)
                frr = jax.lax.dot_general(
                    ar_ref[0, g], acc, (((1,), (1,)), ((), ())),
                    preferred_element_type=jnp.float32)
                fl = fld[:, 0:1]
                fd = fl + fld[:, 1:2]
                d = jnp.maximum(fd, 0.2 * fd)
                b1_scr[g] = fl - d
                b2_scr[g] = 0.2 * fl - d
                fr_scr[g] = frr
                fr2_scr[g] = 0.2 * frr

        madd = m_ref[0].astype(jnp.float32)
        for g in range(H):
            b1 = b1_scr[g, pl.ds(i * BR, BR), :]
            b2 = b2_scr[g, pl.ds(i * BR, BR), :]
            t = jnp.maximum(fr_scr[g] + b1, fr2_scr[g] + b2)
            p = jnp.exp2(t + madd)
            oext = jnp.dot(p, h_scr[g], preferred_element_type=jnp.float32)
            o = oext[:, :Fo] / oext[:, Fo:]
            o_ref[0, :, g * Fo:(g + 1) * Fo] = jnp.maximum(o, 0.0)

    xmap = (lambda b, i: (b, 0, 0)) if Bx == 2 else (lambda b, i: (0, 0, 0))
    return pl.pallas_call(
        body,
        grid=(2, NR),
        in_specs=[
            pl.BlockSpec((1, N, Fin), xmap),
            pl.BlockSpec((1, H, Fin, Fo), lambda b, i: (b, 0, 0, 0)),
            pl.BlockSpec((1, H, Fo, 2), lambda b, i: (b, 0, 0, 0)),
            pl.BlockSpec((1, H, 1, Fo), lambda b, i: (b, 0, 0, 0)),
            pl.BlockSpec((1, BR, N), lambda b, i: (b, i, 0)),
        ],
        out_specs=pl.BlockSpec((1, BR, H * Fo), lambda b, i: (b, i, 0)),
        out_shape=jax.ShapeDtypeStruct((2, N, H * Fo), jnp.float32),
        scratch_shapes=[
            pltpu.VMEM((H, N, Fo + 1), jnp.float32),
            pltpu.VMEM((H, N, 1), jnp.float32),
            pltpu.VMEM((H, N, 1), jnp.float32),
            pltpu.VMEM((H, 1, N), jnp.float32),
            pltpu.VMEM((H, 1, N), jnp.float32),
        ],
    )(x, W, ALR, AR, masks)


def _gat_final(x, masks, W, ALR, AR, wd, bd):
    _, _, Fin = x.shape
    Fo = W.shape[-1]

    def body(x_ref, w_ref, alr_ref, ar_ref, m_ref, wd_ref, wd2_ref, bd_ref,
             o_ref, h_scr, b1_scr, b2_scr, fr_scr, fr2_scr, s_scr):
        bb = pl.program_id(0)
        i = pl.program_id(1)

        @pl.when((bb == 0) & (i == 0))
        def _ones():
            for g in range(H):
                h_scr[g, :, Fo:] = jnp.ones((N, 1), jnp.float32)
            s_scr[...] = jnp.zeros((2, 1, H * Fo), jnp.float32)

        @pl.when(i == 0)
        def _project():
            for g in range(H):
                acc = jnp.dot(x_ref[0], w_ref[0, g],
                              preferred_element_type=jnp.float32)
                h_scr[g, :, :Fo] = acc
                fld = jnp.dot(acc, alr_ref[0, g],
                              preferred_element_type=jnp.float32)
                frr = jax.lax.dot_general(
                    ar_ref[0, g], acc, (((1,), (1,)), ((), ())),
                    preferred_element_type=jnp.float32)
                fl = fld[:, 0:1]
                fd = fl + fld[:, 1:2]
                d = jnp.maximum(fd, 0.2 * fd)
                b1_scr[g] = fl - d
                b2_scr[g] = 0.2 * fl - d
                fr_scr[g] = frr
                fr2_scr[g] = 0.2 * frr

        madd = m_ref[0].astype(jnp.float32)
        cols = []
        for g in range(H):
            b1 = b1_scr[g, pl.ds(i * BR, BR), :]
            b2 = b2_scr[g, pl.ds(i * BR, BR), :]
            t = jnp.maximum(fr_scr[g] + b1, fr2_scr[g] + b2)
            p = jnp.exp2(t + madd)
            oext = jnp.dot(p, h_scr[g], preferred_element_type=jnp.float32)
            o = jnp.maximum(oext[:, :Fo] / oext[:, Fo:], 0.0)
            cols.append(jnp.sum(o, axis=0, keepdims=True))
        part = jnp.concatenate(cols, axis=1)

        @pl.when(bb == 0)
        def _acc0():
            s_scr[0] = s_scr[0] + part

        @pl.when(bb == 1)
        def _acc1():
            s_scr[1] = s_scr[1] + part

        @pl.when((bb == 1) & (i == NR - 1))
        def _flush():
            def tot(z):
                return jnp.sum(z, axis=(0, 1), keepdims=True)
            s0 = s_scr[0]
            s1 = s_scr[1]
            n2 = tot(s0 * s0) + tot(s1 * s1)
            num = tot(s0 * wd_ref[0]) + tot(s1 * wd2_ref[0])
            nrm = jnp.maximum(jnp.sqrt(n2), jnp.float32(1e-12))
            o_ref[...] = num / nrm + bd_ref[...]

    return pl.pallas_call(
        body,
        grid=(2, NR),
        in_specs=[
            pl.BlockSpec((1, N, Fin), lambda b, i: (b, 0, 0)),
            pl.BlockSpec((1, H, Fin, Fo), lambda b, i: (b, 0, 0, 0)),
            pl.BlockSpec((1, H, Fo, 2), lambda b, i: (b, 0, 0, 0)),
            pl.BlockSpec((1, H, 1, Fo), lambda b, i: (b, 0, 0, 0)),
            pl.BlockSpec((1, BR, N), lambda b, i: (b, i, 0)),
            pl.BlockSpec((1, 1, H * Fo), lambda b, i: (0, 0, 0)),
            pl.BlockSpec((1, 1, H * Fo), lambda b, i: (1, 0, 0)),
            pl.BlockSpec((1, 1), lambda b, i: (0, 0)),
        ],
        out_specs=pl.BlockSpec((1, 1), lambda b, i: (0, 0)),
        out_shape=jax.ShapeDtypeStruct((1, 1), jnp.float32),
        scratch_shapes=[
            pltpu.VMEM((H, N, Fo + 1), jnp.float32),
            pltpu.VMEM((H, N, 1), jnp.float32),
            pltpu.VMEM((H, N, 1), jnp.float32),
            pltpu.VMEM((H, 1, N), jnp.float32),
            pltpu.VMEM((H, 1, N), jnp.float32),
            pltpu.VMEM((2, 1, H * Fo), jnp.float32),
        ],
    )(x, W, ALR, AR, masks, wd, wd, bd)


def _att_prep(Ai, An, Fo):
    log2e = jnp.float32(1.4426950408889634)
    A2 = jnp.stack([Ai, An]) * log2e
    alr = jnp.stack([A2[:, :, :Fo], A2[:, :, Fo:]], axis=-1)
    ar = A2[:, :, None, Fo:]
    return alr, ar


def kernel(v, a_int, a_nh, W1i, A1i, W1n, A1n, W2i, A2i, W2n, A2n,
           W3i, A3i, W3n, A3n, Wd, bd):
    masks = _mask_prep(a_int, a_nh)

    alr1, ar1 = _att_prep(A1i, A1n, 16)
    h = _gat(v[None], masks, jnp.stack([W1i, W1n]), alr1, ar1)
    alr2, ar2 = _att_prep(A2i, A2n, 32)
    h = _gat(h, masks, jnp.stack([W2i, W2n]), alr2, ar2)
    alr3, ar3 = _att_prep(A3i, A3n, 64)
    out = _gat_final(h, masks, jnp.stack([W3i, W3n]), alr3, ar3,
                     Wd[:, 0].reshape(2, 1, H * 64), bd.reshape(1, 1))
    return out.reshape(1)
